# baseline (speedup 1.0000x reference)
import jax
import jax.numpy as jnp
from jax.experimental import pallas as pl
from jax.experimental.pallas import tpu as pltpu

_ROWS = 8192
_COLS = 1024
_BLK = 1024
_GRID = _ROWS // _BLK


def _tc_body(src_ref, tar_ref, out_ref, acc_ref):
    i = pl.program_id(0)

    @pl.when(i == 0)
    def _():
        acc_ref[0] = 0.0
        acc_ref[1] = 0.0

    s = src_ref[...]
    t = tar_ref[...]
    mask = t > 0.05
    d = s - t
    sq = jnp.where(mask, d * d, 0.0)
    cnt = jnp.where(mask, 1.0, 0.0)
    acc_ref[0] += jnp.sum(sq)
    acc_ref[1] += jnp.sum(cnt)

    @pl.when(i == _GRID - 1)
    def _():
        out_ref[0, 0] = acc_ref[0] / acc_ref[1]


def kernel(src, tar):
    src2 = src.reshape(_ROWS, _COLS)
    tar2 = tar.reshape(_ROWS, _COLS)
    out = pl.pallas_call(
        _tc_body,
        grid=(_GRID,),
        in_specs=[
            pl.BlockSpec((_BLK, _COLS), lambda i: (i, 0)),
            pl.BlockSpec((_BLK, _COLS), lambda i: (i, 0)),
        ],
        out_specs=pl.BlockSpec(memory_space=pltpu.SMEM),
        out_shape=jax.ShapeDtypeStruct((1, 1), jnp.float32),
        scratch_shapes=[pltpu.SMEM((2,), jnp.float32)],
    )(src2, tar2)
    return out[0, 0]
